# baseline (speedup 1.0000x reference)
import jax
import jax.numpy as jnp
from jax.experimental import pallas as pl
from jax.experimental.pallas import tpu as pltpu

N = 4096
D = 128
CHUNK = 256
NBUF = 4
NSUB = 4
SUBROWS = CHUNK // NSUB
NCHUNKS = N // CHUNK


def _fused_gcn_kernel(bf_ref, bl_ref, w1_ref, b1_ref, a1_ref, w2_ref, b2_ref,
                      a2_ref, adj_hbm, diff_hbm, o_ref, abuf, dbuf, sems):
    f1 = bf_ref[...].astype(jnp.bfloat16)
    f2 = bl_ref[...].astype(jnp.bfloat16)
    a1 = a1_ref[0, 0]
    a2 = a2_ref[0, 0]

    def copy_pair(i):
        slot = i % NBUF
        a_cps = []
        d_cps = []
        for s in range(NSUB):
            rows = pl.ds(i * CHUNK + s * SUBROWS, SUBROWS)
            brows = pl.ds(s * SUBROWS, SUBROWS)
            a_cps.append(pltpu.make_async_copy(
                adj_hbm.at[rows, :], abuf.at[slot].at[brows, :],
                sems.at[0, slot]))
            d_cps.append(pltpu.make_async_copy(
                diff_hbm.at[rows, :], dbuf.at[slot].at[brows, :],
                sems.at[1, slot]))
        return a_cps, d_cps

    def start_pair(i):
        a_cps, d_cps = copy_pair(i)
        for a_cp, d_cp in zip(a_cps, d_cps):
            a_cp.start()
            d_cp.start()

    for i in range(NBUF):
        start_pair(i)

    for i in range(NCHUNKS):
        slot = i % NBUF
        a_cps, d_cps = copy_pair(i)
        for a_cp in a_cps:
            a_cp.wait()
        agg1 = jnp.dot(abuf[slot].astype(jnp.bfloat16), f1,
                       preferred_element_type=jnp.float32)
        for d_cp in d_cps:
            d_cp.wait()
        agg2 = jnp.dot(dbuf[slot].astype(jnp.bfloat16), f2,
                       preferred_element_type=jnp.float32)
        if i + NBUF < NCHUNKS:
            start_pair(i + NBUF)
        t1 = jnp.dot(agg1, w1_ref[...], preferred_element_type=jnp.float32) + b1_ref[...]
        t2 = jnp.dot(agg2, w2_ref[...], preferred_element_type=jnp.float32) + b2_ref[...]
        o_ref[pl.ds(i * CHUNK, CHUNK), :] = (
            jnp.where(t1 >= 0, t1, a1 * t1) + jnp.where(t2 >= 0, t2, a2 * t2))


def kernel(bf, bl, adj, diff, W1, b1, a1, W2, b2, a2):
    adj2 = adj.reshape(N, N)
    diff2 = diff.reshape(N, N)
    bf2 = bf.reshape(N, D)
    bl2 = bl.reshape(N, D)
    b1r = b1.reshape(1, D)
    b2r = b2.reshape(1, D)
    a1r = a1.reshape(1, 1)
    a2r = a2.reshape(1, 1)

    vmem = pl.BlockSpec(memory_space=pltpu.MemorySpace.VMEM)
    hbm = pl.BlockSpec(memory_space=pltpu.MemorySpace.HBM)

    out = pl.pallas_call(
        _fused_gcn_kernel,
        in_specs=[vmem, vmem, vmem, vmem, vmem, vmem, vmem, vmem, hbm, hbm],
        out_specs=vmem,
        out_shape=jax.ShapeDtypeStruct((N, D), jnp.float32),
        scratch_shapes=[
            pltpu.VMEM((NBUF, CHUNK, N), jnp.float32),
            pltpu.VMEM((NBUF, CHUNK, N), jnp.float32),
            pltpu.SemaphoreType.DMA((2, NBUF)),
        ],
    )(bf2, bl2, W1, b1r, a1r, W2, b2r, a2r, adj2, diff2)
    return out.reshape(1, N, D)


# speedup vs baseline: 1.0015x; 1.0015x over previous
import jax
import jax.numpy as jnp
from jax.experimental import pallas as pl
from jax.experimental.pallas import tpu as pltpu

N = 4096
D = 128
CHUNK = 256
NBUF = 8
NCHUNKS = N // CHUNK


def _fused_gcn_kernel(bf_ref, bl_ref, w1_ref, b1_ref, a1_ref, w2_ref, b2_ref,
                      a2_ref, adj_hbm, diff_hbm, o_ref, buf, sems):
    f1 = bf_ref[...].astype(jnp.bfloat16)
    f2 = bl_ref[...].astype(jnp.bfloat16)
    a1 = a1_ref[0, 0]
    a2 = a2_ref[0, 0]

    def make_copy(src_hbm, i, slot):
        return pltpu.make_async_copy(
            src_hbm.at[pl.ds(i * CHUNK, CHUNK), :], buf.at[slot],
            sems.at[slot])

    for i in range(NBUF):
        make_copy(adj_hbm, i, i).start()

    for i in range(NCHUNKS):
        slot = i % NBUF
        make_copy(adj_hbm, i, slot).wait()
        agg1 = jnp.dot(buf[slot].astype(jnp.bfloat16), f1,
                       preferred_element_type=jnp.float32)
        nxt = i + NBUF
        if nxt < NCHUNKS:
            make_copy(adj_hbm, nxt, slot).start()
        else:
            make_copy(diff_hbm, nxt - NCHUNKS, slot).start()
        t1 = jnp.dot(agg1, w1_ref[...], preferred_element_type=jnp.float32) + b1_ref[...]
        o_ref[pl.ds(i * CHUNK, CHUNK), :] = jnp.where(t1 >= 0, t1, a1 * t1)

    for i in range(NCHUNKS):
        slot = (NCHUNKS + i) % NBUF
        make_copy(diff_hbm, i, slot).wait()
        agg2 = jnp.dot(buf[slot].astype(jnp.bfloat16), f2,
                       preferred_element_type=jnp.float32)
        nxt = i + NBUF
        if nxt < NCHUNKS:
            make_copy(diff_hbm, nxt, slot).start()
        t2 = jnp.dot(agg2, w2_ref[...], preferred_element_type=jnp.float32) + b2_ref[...]
        rows = pl.ds(i * CHUNK, CHUNK)
        o_ref[rows, :] = o_ref[rows, :] + jnp.where(t2 >= 0, t2, a2 * t2)


def kernel(bf, bl, adj, diff, W1, b1, a1, W2, b2, a2):
    adj2 = adj.reshape(N, N)
    diff2 = diff.reshape(N, N)
    bf2 = bf.reshape(N, D)
    bl2 = bl.reshape(N, D)
    b1r = b1.reshape(1, D)
    b2r = b2.reshape(1, D)
    a1r = a1.reshape(1, 1)
    a2r = a2.reshape(1, 1)

    vmem = pl.BlockSpec(memory_space=pltpu.MemorySpace.VMEM)
    hbm = pl.BlockSpec(memory_space=pltpu.MemorySpace.HBM)

    out = pl.pallas_call(
        _fused_gcn_kernel,
        in_specs=[vmem, vmem, vmem, vmem, vmem, vmem, vmem, vmem, hbm, hbm],
        out_specs=vmem,
        out_shape=jax.ShapeDtypeStruct((N, D), jnp.float32),
        scratch_shapes=[
            pltpu.VMEM((NBUF, CHUNK, N), jnp.float32),
            pltpu.SemaphoreType.DMA((NBUF,)),
        ],
    )(bf2, bl2, W1, b1r, a1r, W2, b2r, a2r, adj2, diff2)
    return out.reshape(1, N, D)


# speedup vs baseline: 1.0186x; 1.0171x over previous
import jax
import jax.numpy as jnp
from jax.experimental import pallas as pl
from jax.experimental.pallas import tpu as pltpu

N = 4096
D = 128
CHUNK = 256
NBUF = 8
NCHUNKS = N // CHUNK


def _fused_gcn_kernel(w1_ref, b1_ref, a1_ref, w2_ref, b2_ref, a2_ref,
                      bf_hbm, bl_hbm, adj_hbm, diff_hbm, o_hbm,
                      buf, fbuf, acc, sems, fsems, osem):
    a1 = a1_ref[0, 0]
    a2 = a2_ref[0, 0]

    def make_copy(src_hbm, i, slot):
        return pltpu.make_async_copy(
            src_hbm.at[pl.ds(i * CHUNK, CHUNK), :], buf.at[slot],
            sems.at[slot])

    f1_cp = pltpu.make_async_copy(bf_hbm, fbuf.at[0], fsems.at[0])
    f2_cp = pltpu.make_async_copy(bl_hbm, fbuf.at[1], fsems.at[1])
    f1_cp.start()
    f2_cp.start()

    for i in range(NBUF):
        make_copy(adj_hbm, i, i).start()

    f1_cp.wait()
    f1 = fbuf[0].astype(jnp.bfloat16)
    f2_cp.wait()
    f2 = fbuf[1].astype(jnp.bfloat16)

    for i in range(NCHUNKS):
        slot = i % NBUF
        make_copy(adj_hbm, i, slot).wait()
        agg1 = jnp.dot(buf[slot].astype(jnp.bfloat16), f1,
                       preferred_element_type=jnp.float32)
        nxt = i + NBUF
        if nxt < NCHUNKS:
            make_copy(adj_hbm, nxt, slot).start()
        else:
            make_copy(diff_hbm, nxt - NCHUNKS, slot).start()
        t1 = jnp.dot(agg1, w1_ref[...], preferred_element_type=jnp.float32) + b1_ref[...]
        acc[pl.ds(i * CHUNK, CHUNK), :] = jnp.where(t1 >= 0, t1, a1 * t1)

    for i in range(NCHUNKS):
        slot = (NCHUNKS + i) % NBUF
        make_copy(diff_hbm, i, slot).wait()
        agg2 = jnp.dot(buf[slot].astype(jnp.bfloat16), f2,
                       preferred_element_type=jnp.float32)
        nxt = i + NBUF
        if nxt < NCHUNKS:
            make_copy(diff_hbm, nxt, slot).start()
        t2 = jnp.dot(agg2, w2_ref[...], preferred_element_type=jnp.float32) + b2_ref[...]
        rows = pl.ds(i * CHUNK, CHUNK)
        acc[rows, :] = acc[rows, :] + jnp.where(t2 >= 0, t2, a2 * t2)
        pltpu.make_async_copy(acc.at[rows, :], o_hbm.at[rows, :], osem).start()

    for i in range(NCHUNKS):
        rows = pl.ds(i * CHUNK, CHUNK)
        pltpu.make_async_copy(acc.at[rows, :], o_hbm.at[rows, :], osem).wait()


def kernel(bf, bl, adj, diff, W1, b1, a1, W2, b2, a2):
    adj2 = adj.reshape(N, N)
    diff2 = diff.reshape(N, N)
    bf2 = bf.reshape(N, D)
    bl2 = bl.reshape(N, D)
    b1r = b1.reshape(1, D)
    b2r = b2.reshape(1, D)
    a1r = a1.reshape(1, 1)
    a2r = a2.reshape(1, 1)

    vmem = pl.BlockSpec(memory_space=pltpu.MemorySpace.VMEM)
    hbm = pl.BlockSpec(memory_space=pltpu.MemorySpace.HBM)

    out = pl.pallas_call(
        _fused_gcn_kernel,
        in_specs=[vmem, vmem, vmem, vmem, vmem, vmem, hbm, hbm, hbm, hbm],
        out_specs=hbm,
        out_shape=jax.ShapeDtypeStruct((N, D), jnp.float32),
        scratch_shapes=[
            pltpu.VMEM((NBUF, CHUNK, N), jnp.float32),
            pltpu.VMEM((2, N, D), jnp.float32),
            pltpu.VMEM((N, D), jnp.float32),
            pltpu.SemaphoreType.DMA((NBUF,)),
            pltpu.SemaphoreType.DMA((2,)),
            pltpu.SemaphoreType.DMA,
        ],
    )(W1, b1r, a1r, W2, b2r, a2r, bf2, bl2, adj2, diff2)
    return out.reshape(1, N, D)
